# baseline (speedup 1.0000x reference)
import functools
import math

import jax
import jax.numpy as jnp
from jax import lax
from jax.experimental import pallas as pl
from jax.experimental.pallas import tpu as pltpu
from jax.experimental.pallas import tpu_sc as plsc

D_MODEL = 64
D_PAD = 128
SCALE = math.sqrt(D_MODEL)

NUM_CORES = 2
NUM_SUBCORES = 16
NUM_WORKERS = NUM_CORES * NUM_SUBCORES
LANES = 16

IDX_SPLITS = ((0, 128), (128, 72))


@functools.lru_cache(maxsize=None)
def _build(batch: int, seq: int):
    rows_per_w = batch // NUM_WORKERS

    mesh = plsc.VectorSubcoreMesh(core_axis_name="c", subcore_axis_name="s")

    @functools.partial(
        pl.kernel,
        mesh=mesh,
        out_type=jax.ShapeDtypeStruct((batch, seq, D_MODEL), jnp.float32),
        scratch_types=[
            pltpu.VMEM((rows_per_w, seq), jnp.int32),
            pltpu.VMEM((rows_per_w, seq), jnp.int32),
            pltpu.VMEM((seq, D_PAD), jnp.float32),
            pltpu.VMEM((seq, D_PAD), jnp.float32),
            pltpu.VMEM((1, seq, D_MODEL), jnp.float32),
            pltpu.SemaphoreType.DMA,
            pltpu.SemaphoreType.DMA,
        ],
        compiler_params=pltpu.CompilerParams(use_tc_tiling_on_sc=True),
    )
    def emb_kernel(x_hbm, tbl_hbm, out_hbm, idx_v, idxq_v, buf0, buf1, cbuf,
                   sem0, sem1):
        wid = lax.axis_index("s") * NUM_CORES + lax.axis_index("c")
        row0 = wid * rows_per_w

        bufs = (buf0, buf1)
        sems = (sem0, sem1)

        pltpu.sync_copy(x_hbm.at[pl.ds(row0, rows_per_w)], idx_v)

        def halve(r, carry):
            def body(k, c2):
                sl = pl.ds(k * LANES, LANES)
                idxq_v[r, sl] = lax.shift_right_logical(idx_v[r, sl], 1)
                return c2

            lax.fori_loop(0, seq // LANES, body, 0)
            if seq % LANES:
                sl = pl.ds(seq - LANES, LANES)
                idxq_v[r, sl] = lax.shift_right_logical(idx_v[r, sl], 1)
            return carry

        lax.fori_loop(0, rows_per_w, halve, 0)

        def fire(r, buf, sem):
            waits = []
            for off, n in IDX_SPLITS:
                waits.append(
                    pltpu.async_copy(
                        tbl_hbm.at[idxq_v.at[r, pl.ds(off, n)]],
                        buf.at[pl.ds(off, n)],
                        sem,
                    )
                )
            return waits

        def select(r, buf):
            def half_group(s0, lanes):
                par = (idx_v[r, pl.ds(s0, LANES)] & 1) * D_MODEL
                for j in lanes:
                    base = par[j]
                    for q in range(D_MODEL // LANES):
                        v = buf[s0 + j, pl.ds(base + q * LANES, LANES)]
                        cbuf[0, s0 + j, pl.ds(q * LANES, LANES)] = v * SCALE

            def body(k, carry):
                half_group(k * LANES, range(LANES))
                return carry

            lax.fori_loop(0, seq // LANES, body, 0)
            tail = seq % LANES
            if tail:
                half_group(seq - LANES, range(LANES - tail, LANES))

        def drain(buf, sem):
            for off, n in IDX_SPLITS:
                pltpu.make_async_copy(
                    tbl_hbm.at[idxq_v.at[0, pl.ds(off, n)]],
                    buf.at[pl.ds(off, n)],
                    sem,
                ).wait()

        fire(0, buf0, sem0)

        def pair(p, carry):
            r0 = 2 * p
            fire(r0 + 1, buf1, sem1)
            drain(buf0, sem0)
            select(r0, buf0)
            pltpu.sync_copy(cbuf, out_hbm.at[pl.ds(row0 + r0, 1)])

            @pl.when(r0 + 2 < rows_per_w)
            def _():
                fire(r0 + 2, buf0, sem0)

            drain(buf1, sem1)
            select(r0 + 1, buf1)
            pltpu.sync_copy(cbuf, out_hbm.at[pl.ds(row0 + r0 + 1, 1)])
            return carry

        lax.fori_loop(0, rows_per_w // 2, pair, 0)

    return emb_kernel


def kernel(x, lut):
    batch, seq = x.shape
    vocab, d = lut.shape
    tbl = jnp.reshape(lut, (vocab // 2, 2 * d))
    return _build(batch, seq)(x.astype(jnp.int32), tbl)


# speedup vs baseline: 1.6659x; 1.6659x over previous
import functools
import math

import jax
import jax.numpy as jnp
from jax import lax
from jax.experimental import pallas as pl
from jax.experimental.pallas import tpu as pltpu
from jax.experimental.pallas import tpu_sc as plsc

D_MODEL = 64
SCALE = math.sqrt(D_MODEL)

NUM_CORES = 2
NUM_SUBCORES = 16
NUM_WORKERS = NUM_CORES * NUM_SUBCORES
LANES = 16

ROWS_PER_CHUNK = 2


@functools.lru_cache(maxsize=None)
def _build(batch: int, seq: int):
    rows_per_w = batch // NUM_WORKERS
    num_chunks = rows_per_w // ROWS_PER_CHUNK

    mesh = plsc.VectorSubcoreMesh(core_axis_name="c", subcore_axis_name="s")

    @functools.partial(
        pl.kernel,
        mesh=mesh,
        out_type=jax.ShapeDtypeStruct((batch, seq, D_MODEL), jnp.float32),
        scratch_types=[
            pltpu.VMEM((rows_per_w, seq), jnp.int32),
            pltpu.VMEM((ROWS_PER_CHUNK, seq, D_MODEL), jnp.float32),
            pltpu.VMEM((ROWS_PER_CHUNK, seq, D_MODEL), jnp.float32),
            pltpu.SemaphoreType.DMA,
            pltpu.SemaphoreType.DMA,
        ],
        compiler_params=pltpu.CompilerParams(use_tc_tiling_on_sc=True),
    )
    def emb_kernel(x_hbm, lut_hbm, out_hbm, idx_v, buf0, buf1, sem0, sem1):
        wid = lax.axis_index("s") * NUM_CORES + lax.axis_index("c")
        row0 = wid * rows_per_w

        bufs = (buf0, buf1)
        sems = (sem0, sem1)

        pltpu.sync_copy(x_hbm.at[pl.ds(row0, rows_per_w)], idx_v)

        def issue(g, buf, sem):
            def fetch16(r, s0, v, lanes):
                for j in lanes:
                    pltpu.async_copy(
                        lut_hbm.at[v[j]],
                        buf.at[r, s0 + j, pl.ds(0, D_MODEL)],
                        sem,
                    )

            for r in range(ROWS_PER_CHUNK):
                xrow = g * ROWS_PER_CHUNK + r

                def body(k, carry, r=r, xrow=xrow):
                    v = idx_v[xrow, pl.ds(k * LANES, LANES)]
                    fetch16(r, k * LANES, v, range(LANES))
                    return carry

                lax.fori_loop(0, seq // LANES, body, 0)
                tail = seq % LANES
                if tail:
                    v = idx_v[xrow, pl.ds(seq - LANES, LANES)]
                    fetch16(r, seq - LANES, v, range(LANES - tail, LANES))

        def drain(buf, sem):
            pltpu.make_async_copy(
                out_hbm.at[pl.ds(0, ROWS_PER_CHUNK)], buf, sem
            ).wait()

        def scale(buf):
            def body(s, carry):
                for r in range(ROWS_PER_CHUNK):
                    for c in range(D_MODEL // LANES):
                        sl = pl.ds(c * LANES, LANES)
                        buf[r, s, sl] = buf[r, s, sl] * SCALE
                return carry

            lax.fori_loop(0, seq, body, 0, unroll=4)

        issue(0, bufs[0], sems[0])
        for g in range(num_chunks):
            if g + 1 < num_chunks:
                issue(g + 1, bufs[(g + 1) % 2], sems[(g + 1) % 2])
            drain(bufs[g % 2], sems[g % 2])
            scale(bufs[g % 2])
            pltpu.sync_copy(
                bufs[g % 2],
                out_hbm.at[pl.ds(row0 + g * ROWS_PER_CHUNK, ROWS_PER_CHUNK)],
            )

    return emb_kernel


def kernel(x, lut):
    batch, seq = x.shape
    return _build(batch, seq)(x.astype(jnp.int32), lut)
